# baseline (speedup 1.0000x reference)
import functools

import jax
import jax.numpy as jnp
from jax import lax
from jax.experimental import pallas as pl
from jax.experimental.pallas import tpu as pltpu
from jax.experimental.pallas import tpu_sc as plsc

_SEQ_BLOCK = 1024
_D = 768


def _tc_body(x_ref, p_ref, o_ref):
    o_ref[...] = x_ref[...] + p_ref[...]


def _tc_add(inputs, pos_table):
    batch, seq, dim = inputs.shape
    nblk = seq // _SEQ_BLOCK
    return pl.pallas_call(
        _tc_body,
        grid=(nblk,),
        in_specs=[
            pl.BlockSpec((batch, _SEQ_BLOCK, dim), lambda i: (0, i, 0)),
            pl.BlockSpec((_SEQ_BLOCK, dim), lambda i: (i, 0)),
        ],
        out_specs=pl.BlockSpec((batch, _SEQ_BLOCK, dim), lambda i: (0, i, 0)),
        out_shape=jax.ShapeDtypeStruct((batch, seq, dim), inputs.dtype),
    )(inputs, pos_table)


_NC, _NS = 2, 16
_NW = _NC * _NS
_SC_CHUNK = 32


def _sc_body(total_rows, seq, x_hbm, p_hbm, o_hbm, xbuf, pbuf):
    wid = lax.axis_index("s") * _NC + lax.axis_index("c")
    rows_per_w = total_rows // _NW
    row0 = wid * rows_per_w
    srow0 = lax.rem(row0, seq)
    ce = _SC_CHUNK * _D

    def chunk_body(i, carry):
        xoff = (row0 + i * _SC_CHUNK) * _D
        poff = (srow0 + i * _SC_CHUNK) * _D
        pltpu.sync_copy(x_hbm.at[pl.ds(xoff, ce)], xbuf)
        pltpu.sync_copy(p_hbm.at[pl.ds(poff, ce)], pbuf)

        def vec_body(k, c2):
            for j in range(8):
                o = (k * 8 + j) * 16
                xbuf[pl.ds(o, 16)] = xbuf[pl.ds(o, 16)] + pbuf[pl.ds(o, 16)]
            return c2

        lax.fori_loop(0, ce // 128, vec_body, 0)
        pltpu.sync_copy(xbuf, o_hbm.at[pl.ds(xoff, ce)])
        return carry

    lax.fori_loop(0, rows_per_w // _SC_CHUNK, chunk_body, 0)


def _sc_add(inputs, pos_table):
    batch, seq, dim = inputs.shape
    total_rows = batch * seq
    xf = inputs.reshape(total_rows * dim)
    pf = pos_table.reshape(seq * dim)
    mesh = plsc.VectorSubcoreMesh(core_axis_name="c", subcore_axis_name="s")
    out = pl.kernel(
        functools.partial(_sc_body, total_rows, seq),
        mesh=mesh,
        out_type=jax.ShapeDtypeStruct((total_rows * dim,), jnp.float32),
        scratch_types=[
            pltpu.VMEM((_SC_CHUNK * _D,), jnp.float32),
            pltpu.VMEM((_SC_CHUNK * _D,), jnp.float32),
        ],
    )(xf, pf)
    return out.reshape(batch, seq, dim)


def kernel(inputs, pos_table):
    return _sc_add(inputs, pos_table)


# speedup vs baseline: 1.4397x; 1.4397x over previous
import functools

import jax
import jax.numpy as jnp
from jax import lax
from jax.experimental import pallas as pl
from jax.experimental.pallas import tpu as pltpu
from jax.experimental.pallas import tpu_sc as plsc

_SEQ_BLOCK = 1024
_D = 768


def _tc_body(x_ref, p_ref, o_ref):
    o_ref[...] = x_ref[...] + p_ref[...]


def _tc_add(inputs, pos_table, nbatch=None):
    batch, seq, dim = inputs.shape
    if nbatch is None:
        nbatch = batch
    nblk = seq // _SEQ_BLOCK
    return pl.pallas_call(
        _tc_body,
        grid=(nblk,),
        in_specs=[
            pl.BlockSpec((nbatch, _SEQ_BLOCK, dim), lambda i: (0, i, 0)),
            pl.BlockSpec((_SEQ_BLOCK, dim), lambda i: (i, 0)),
        ],
        out_specs=pl.BlockSpec((nbatch, _SEQ_BLOCK, dim), lambda i: (0, i, 0)),
        out_shape=jax.ShapeDtypeStruct((nbatch, seq, dim), inputs.dtype),
    )(inputs, pos_table)


_NC, _NS = 2, 16
_NW = _NC * _NS
_SC_CHUNK = 32


def _sc_body(row_base, total_rows, seq, x_hbm, p_hbm, o_hbm, xbuf, pbuf):
    wid = lax.axis_index("s") * _NC + lax.axis_index("c")
    rows_per_w = total_rows // _NW
    row0 = row_base + wid * rows_per_w
    srow0 = lax.rem(row0, seq)
    ce = _SC_CHUNK * _D

    def chunk_body(i, carry):
        xoff = (row0 + i * _SC_CHUNK) * _D
        ooff = (row0 - row_base + i * _SC_CHUNK) * _D
        poff = (srow0 + i * _SC_CHUNK) * _D
        pltpu.sync_copy(x_hbm.at[pl.ds(xoff, ce)], xbuf)
        pltpu.sync_copy(p_hbm.at[pl.ds(poff, ce)], pbuf)

        def vec_body(k, c2):
            for j in range(8):
                o = (k * 8 + j) * 16
                xbuf[pl.ds(o, 16)] = xbuf[pl.ds(o, 16)] + pbuf[pl.ds(o, 16)]
            return c2

        lax.fori_loop(0, ce // 128, vec_body, 0)
        pltpu.sync_copy(xbuf, o_hbm.at[pl.ds(ooff, ce)])
        return carry

    lax.fori_loop(0, rows_per_w // _SC_CHUNK, chunk_body, 0)


def _sc_add(inputs, pos_table, batch_base=0, nbatch=None):
    batch, seq, dim = inputs.shape
    if nbatch is None:
        nbatch = batch
    row_base = batch_base * seq
    sc_rows = nbatch * seq
    xf = inputs.reshape(batch * seq * dim)
    pf = pos_table.reshape(seq * dim)
    mesh = plsc.VectorSubcoreMesh(core_axis_name="c", subcore_axis_name="s")
    out = pl.kernel(
        functools.partial(_sc_body, row_base, sc_rows, seq),
        mesh=mesh,
        out_type=jax.ShapeDtypeStruct((sc_rows * dim,), jnp.float32),
        scratch_types=[
            pltpu.VMEM((_SC_CHUNK * _D,), jnp.float32),
            pltpu.VMEM((_SC_CHUNK * _D,), jnp.float32),
        ],
    )(xf, pf)
    return out.reshape(nbatch, seq, dim)


def kernel(inputs, pos_table):
    batch, seq, dim = inputs.shape
    tc_out = _tc_add(inputs, pos_table, nbatch=batch - 1)
    sc_out = _sc_add(inputs, pos_table, batch_base=batch - 1, nbatch=1)
    return jnp.concatenate([tc_out, sc_out], axis=0)


# speedup vs baseline: 6.2524x; 4.3429x over previous
import functools

import jax
import jax.numpy as jnp
from jax import lax
from jax.experimental import pallas as pl
from jax.experimental.pallas import tpu as pltpu
from jax.experimental.pallas import tpu_sc as plsc

_SEQ_BLOCK = 1024
_D = 768


def _tc_body(x_ref, p_ref, o_ref):
    o_ref[...] = x_ref[...] + p_ref[...]


def _tc_add(inputs, pos_table, nbatch=None):
    batch, seq, dim = inputs.shape
    if nbatch is None:
        nbatch = batch
    nblk = seq // _SEQ_BLOCK
    return pl.pallas_call(
        _tc_body,
        grid=(nblk,),
        in_specs=[
            pl.BlockSpec((nbatch, _SEQ_BLOCK, dim), lambda i: (0, i, 0)),
            pl.BlockSpec((_SEQ_BLOCK, dim), lambda i: (i, 0)),
        ],
        out_specs=pl.BlockSpec((nbatch, _SEQ_BLOCK, dim), lambda i: (0, i, 0)),
        out_shape=jax.ShapeDtypeStruct((nbatch, seq, dim), inputs.dtype),
    )(inputs, pos_table)


_NC, _NS = 2, 16
_NW = _NC * _NS
_SC_CHUNK = 32


def _sc_body(row_base, total_rows, seq, x_hbm, p_hbm, o_hbm, xbuf, pbuf):
    wid = lax.axis_index("s") * _NC + lax.axis_index("c")
    rows_per_w = total_rows // _NW
    row0 = row_base + wid * rows_per_w
    srow0 = lax.rem(row0, seq)
    ce = _SC_CHUNK * _D

    def chunk_body(i, carry):
        xoff = (row0 + i * _SC_CHUNK) * _D
        ooff = (row0 - row_base + i * _SC_CHUNK) * _D
        poff = (srow0 + i * _SC_CHUNK) * _D
        pltpu.sync_copy(x_hbm.at[pl.ds(xoff, ce)], xbuf)
        pltpu.sync_copy(p_hbm.at[pl.ds(poff, ce)], pbuf)

        def vec_body(k, c2):
            for j in range(8):
                o = (k * 8 + j) * 16
                xbuf[pl.ds(o, 16)] = xbuf[pl.ds(o, 16)] + pbuf[pl.ds(o, 16)]
            return c2

        lax.fori_loop(0, ce // 128, vec_body, 0)
        pltpu.sync_copy(xbuf, o_hbm.at[pl.ds(ooff, ce)])
        return carry

    lax.fori_loop(0, rows_per_w // _SC_CHUNK, chunk_body, 0)


def _sc_add(inputs, pos_table, batch_base=0, nbatch=None):
    batch, seq, dim = inputs.shape
    if nbatch is None:
        nbatch = batch
    row_base = batch_base * seq
    sc_rows = nbatch * seq
    xf = inputs.reshape(batch * seq * dim)
    pf = pos_table.reshape(seq * dim)
    mesh = plsc.VectorSubcoreMesh(core_axis_name="c", subcore_axis_name="s")
    out = pl.kernel(
        functools.partial(_sc_body, row_base, sc_rows, seq),
        mesh=mesh,
        out_type=jax.ShapeDtypeStruct((sc_rows * dim,), jnp.float32),
        scratch_types=[
            pltpu.VMEM((_SC_CHUNK * _D,), jnp.float32),
            pltpu.VMEM((_SC_CHUNK * _D,), jnp.float32),
        ],
    )(xf, pf)
    return out.reshape(nbatch, seq, dim)


def kernel(inputs, pos_table):
    return _tc_add(inputs, pos_table)
